# baseline (speedup 1.0000x reference)
import functools

import jax
import jax.numpy as jnp
from jax import lax
from jax.experimental import pallas as pl
from jax.experimental.pallas import tpu as pltpu
from jax.experimental.pallas import tpu_sc as plsc

_FEAT = 768
_HID = 256
_K = 8192
_NS = 10
_MOM = 0.999
_N = 4608

_BT_A = 512
_BT_B = 128
_BK = 1024

_NW = 32
_TW = _N // _NW
_C = 16
_NCH = _TW // _C


def _prep_body(cent_ref, cn_ref, c2_ref):
    c = cent_ref[...]
    nrm = jnp.sqrt(jnp.sum(c * c, axis=1, keepdims=True))
    cn = c / (nrm + 1e-12)
    cn_ref[...] = cn
    c2_ref[...] = jnp.sum(cn * cn, axis=1, keepdims=True)


def _head_body(xo_ref, xa_ref, w1_ref, b1_ref, w2_ref, b2_ref,
               ew1_ref, eb1_ref, ew2_ref, eb2_ref,
               z1_ref, n1_ref, z2_ref, loss_ref):
    i = pl.program_id(0)
    w1 = w1_ref[...]
    b1 = b1_ref[...]
    w2 = w2_ref[...]
    b2 = b2_ref[...]
    xo = xo_ref[...]
    h1 = jnp.maximum(jnp.dot(xo, w1, preferred_element_type=jnp.float32) + b1, 0.0)
    z1 = jnp.dot(h1, w2, preferred_element_type=jnp.float32) + b2
    z1_ref[...] = z1
    nr1 = jnp.sqrt(jnp.sum(z1 * z1, axis=1, keepdims=True))
    n1 = z1 / (nr1 + 1e-12)
    n1_ref[...] = n1
    z2_ref[...] = jnp.sum(n1 * n1, axis=1, keepdims=True)
    uw1 = ew1_ref[...] * _MOM + w1 * (1.0 - _MOM)
    ub1 = eb1_ref[...] * _MOM + b1 * (1.0 - _MOM)
    uw2 = ew2_ref[...] * _MOM + w2 * (1.0 - _MOM)
    ub2 = eb2_ref[...] * _MOM + b2 * (1.0 - _MOM)
    xa = xa_ref[...]
    h2 = jnp.maximum(jnp.dot(xa, uw1, preferred_element_type=jnp.float32) + ub1, 0.0)
    za = jnp.dot(h2, uw2, preferred_element_type=jnp.float32) + ub2
    nr2 = jnp.sqrt(jnp.sum(za * za, axis=1, keepdims=True))
    n2 = za / (nr2 + 1e-12)
    d = n1 - n2
    part = jnp.sum(d * d).reshape(1, 1)
    @pl.when(i == 0)
    def _():
        loss_ref[...] = jnp.zeros((1, 1), jnp.float32)
    loss_ref[...] += part
    @pl.when(i == pl.num_programs(0) - 1)
    def _():
        loss_ref[...] = loss_ref[...] / float(_N * _HID)


def _vq_body(n1_ref, z2_ref, cn_ref, c2_ref, idx_ref):
    mm = lax.dot_general(n1_ref[...], cn_ref[...], (((1,), (1,)), ((), ())),
                         preferred_element_type=jnp.float32)
    dist = (z2_ref[...] + c2_ref[...]) - 2.0 * mm
    dmin = jnp.min(dist, axis=1, keepdims=True)
    ii = lax.broadcasted_iota(jnp.int32, dist.shape, 1)
    cand = jnp.where(dist == dmin, ii, _K)
    idx_ref[...] = jnp.min(cand, axis=1, keepdims=True)


@functools.cache
def _make_sc_gather():
    @functools.partial(
        pl.kernel,
        out_type=[jax.ShapeDtypeStruct((_N, _NS * _HID), jnp.float32),
                  jax.ShapeDtypeStruct((_N, _HID), jnp.float32)],
        mesh=plsc.VectorSubcoreMesh(core_axis_name="c", subcore_axis_name="s"),
        scratch_types=[
            pltpu.VMEM((_TW,), jnp.int32),
            pltpu.VMEM((2, _C, _NS * _HID), jnp.float32),
            pltpu.VMEM((2, _C, _HID), jnp.float32),
            pltpu.SemaphoreType.DMA,
            pltpu.SemaphoreType.DMA,
        ],
    )
    def _sc_gather(queue_hbm, cent_hbm, idx_hbm, z1_hbm, pos_hbm, proxy_hbm,
                   idx_v, qbuf, cbuf, semq, semc):
        wid = lax.axis_index("s") * 2 + lax.axis_index("c")
        base = wid * _TW
        pltpu.sync_copy(idx_hbm.at[pl.ds(base, _TW)], idx_v)
        for j in range(_NCH):
            b = j % 2
            tb = base + j * _C
            row = idx_v.at[pl.ds(j * _C, _C)]
            cq = pltpu.async_copy(queue_hbm.at[row], qbuf.at[b], semq)
            cc = pltpu.async_copy(cent_hbm.at[row], cbuf.at[b], semc)
            cq.wait()
            cc.wait()
            pltpu.sync_copy(z1_hbm.at[pl.ds(tb, _C)],
                            qbuf.at[b, :, pl.ds((_NS - 1) * _HID, _HID)])
            pltpu.sync_copy(qbuf.at[b], pos_hbm.at[pl.ds(tb, _C)])
            pltpu.sync_copy(cbuf.at[b], proxy_hbm.at[pl.ds(tb, _C)])

    return _sc_gather


def kernel(img, aug_img, W1, b1, W2, b2, eW1, eb1, eW2, eb2, centroid, queue):
    xo = jnp.transpose(img, (0, 2, 3, 1)).reshape(-1, _FEAT)
    xa = jnp.transpose(aug_img, (0, 2, 3, 1)).reshape(-1, _FEAT)
    b1r = b1.reshape(1, _FEAT)
    b2r = b2.reshape(1, _HID)
    eb1r = eb1.reshape(1, _FEAT)
    eb2r = eb2.reshape(1, _HID)

    cn, c2 = pl.pallas_call(
        _prep_body,
        grid=(_K // _BK,),
        in_specs=[pl.BlockSpec((_BK, _HID), lambda i: (i, 0))],
        out_specs=[pl.BlockSpec((_BK, _HID), lambda i: (i, 0)),
                   pl.BlockSpec((_BK, 1), lambda i: (i, 0))],
        out_shape=[jax.ShapeDtypeStruct((_K, _HID), jnp.float32),
                   jax.ShapeDtypeStruct((_K, 1), jnp.float32)],
    )(centroid)

    z1, n1, z2, loss = pl.pallas_call(
        _head_body,
        grid=(_N // _BT_A,),
        in_specs=[pl.BlockSpec((_BT_A, _FEAT), lambda i: (i, 0)),
                  pl.BlockSpec((_BT_A, _FEAT), lambda i: (i, 0)),
                  pl.BlockSpec((_FEAT, _FEAT), lambda i: (0, 0)),
                  pl.BlockSpec((1, _FEAT), lambda i: (0, 0)),
                  pl.BlockSpec((_FEAT, _HID), lambda i: (0, 0)),
                  pl.BlockSpec((1, _HID), lambda i: (0, 0)),
                  pl.BlockSpec((_FEAT, _FEAT), lambda i: (0, 0)),
                  pl.BlockSpec((1, _FEAT), lambda i: (0, 0)),
                  pl.BlockSpec((_FEAT, _HID), lambda i: (0, 0)),
                  pl.BlockSpec((1, _HID), lambda i: (0, 0))],
        out_specs=[pl.BlockSpec((_BT_A, _HID), lambda i: (i, 0)),
                   pl.BlockSpec((_BT_A, _HID), lambda i: (i, 0)),
                   pl.BlockSpec((_BT_A, 1), lambda i: (i, 0)),
                   pl.BlockSpec((1, 1), lambda i: (0, 0))],
        out_shape=[jax.ShapeDtypeStruct((_N, _HID), jnp.float32),
                   jax.ShapeDtypeStruct((_N, _HID), jnp.float32),
                   jax.ShapeDtypeStruct((_N, 1), jnp.float32),
                   jax.ShapeDtypeStruct((1, 1), jnp.float32)],
    )(xo, xa, W1, b1r, W2, b2r, eW1, eb1r, eW2, eb2r)

    idx = pl.pallas_call(
        _vq_body,
        grid=(_N // _BT_B,),
        in_specs=[pl.BlockSpec((_BT_B, _HID), lambda i: (i, 0)),
                  pl.BlockSpec((_BT_B, 1), lambda i: (i, 0)),
                  pl.BlockSpec((_K, _HID), lambda i: (0, 0)),
                  pl.BlockSpec((1, _K), lambda i: (0, 0))],
        out_specs=pl.BlockSpec((_BT_B, 1), lambda i: (i, 0)),
        out_shape=jax.ShapeDtypeStruct((_N, 1), jnp.int32),
    )(n1, z2, cn, c2.reshape(1, _K))

    pos2d, pos_proxy = _make_sc_gather()(
        queue.reshape(_K, _NS * _HID), centroid, idx.reshape(_N), z1)
    positives = pos2d.reshape(_N, _NS, _HID)

    out = jnp.transpose(n1.reshape(8, 24, 24, _HID), (0, 3, 1, 2))
    loss1 = loss[0, 0]
    return (out, pos_proxy, positives, loss1)
